# Initial kernel scaffold; baseline (speedup 1.0000x reference)
import jax
import jax.numpy as jnp
from jax.experimental import pallas as pl


def kernel(content, edge_index, emb, W_proj, b_proj, W0, b0, W1, b1):
    raise NotImplementedError("write your pallas kernel here")


# speedup vs baseline: 4.6893x; 4.6893x over previous
import dataclasses
import functools

import jax
import jax.numpy as jnp
from jax import lax
from jax.experimental import pallas as pl
from jax.experimental.pallas import tpu as pltpu
from jax.experimental.pallas import tpu_sc as plsc

N = 10000
E = 320000
D = 128
NC = 2
NS = 16
NW = NC * NS
EPW = E // NW
K = 80
CH = EPW // K
SEG = 5
EPS = EPW // SEG
CHS = CH // SEG
NP = 10240
RPT = NP // NS
RB = 1024
GRID = NP // RB
DW = 16


def _leaky(x):
    return jnp.where(x >= 0, x, 0.01 * x)


def _embed_body(c_ref, e_ref, wp_ref, bp_ref, o_ref):
    y = lax.dot_general(c_ref[...], wp_ref[...], (((1,), (1,)), ((), ())),
                        preferred_element_type=jnp.float32) + bp_ref[...]
    o_ref[...] = e_ref[...] + _leaky(y)


def _embed(content_p, emb1_p, wp, bp):
    return pl.pallas_call(
        _embed_body,
        grid=(GRID,),
        in_specs=[
            pl.BlockSpec((RB, D), lambda i: (i, 0)),
            pl.BlockSpec((RB, D), lambda i: (i, 0)),
            pl.BlockSpec((D, D), lambda i: (0, 0)),
            pl.BlockSpec((1, D), lambda i: (0, 0)),
        ],
        out_specs=pl.BlockSpec((RB, D), lambda i: (i, 0)),
        out_shape=jax.ShapeDtypeStruct((NP, D), jnp.float32),
    )(content_p, emb1_p, wp, bp)


def _layer_body(relu, h_ref, agg_ref, deg_ref, w_ref, b_ref, o_ref):
    s = agg_ref[0] + agg_ref[1]
    dcol = (deg_ref[0] + deg_ref[1])[:, 0:1]
    inv = 1.0 / jnp.maximum(dcol, 1.0)
    h = h_ref[...]
    h_agg = s * inv
    w = w_ref[...]
    y = lax.dot_general(h, w[:, :D], (((1,), (1,)), ((), ())),
                        preferred_element_type=jnp.float32)
    y = y + lax.dot_general(h_agg, w[:, D:], (((1,), (1,)), ((), ())),
                            preferred_element_type=jnp.float32)
    y = y + b_ref[...]
    if relu:
        y = _leaky(y)
    nrm = jnp.maximum(jnp.sqrt(jnp.sum(y * y, axis=1, keepdims=True)), 1e-6)
    o_ref[...] = y / nrm


def _layer(h, agg, deg, w, b, relu):
    return pl.pallas_call(
        functools.partial(_layer_body, relu),
        grid=(GRID,),
        in_specs=[
            pl.BlockSpec((RB, D), lambda i: (i, 0)),
            pl.BlockSpec((NC, RB, D), lambda i: (0, i, 0)),
            pl.BlockSpec((NC, RB, D), lambda i: (0, i, 0)),
            pl.BlockSpec((D, 2 * D), lambda i: (0, 0)),
            pl.BlockSpec((1, D), lambda i: (0, 0)),
        ],
        out_specs=pl.BlockSpec((RB, D), lambda i: (i, 0)),
        out_shape=jax.ShapeDtypeStruct((NP, D), jnp.float32),
    )(h, agg, deg, w, b)


def _sc_mesh():
    return plsc.VectorSubcoreMesh(core_axis_name="c", subcore_axis_name="s")


def _sc_params():
    return pltpu.CompilerParams()


def _sc_scatter(h, src, dst3, with_deg):
    z = jnp.zeros((RPT, D), jnp.float32)

    out_type = [jax.ShapeDtypeStruct((NC, NP, D), jnp.float32)]
    scratch = [
        pltpu.VMEM_SHARED((NP, D), jnp.float32),
        pltpu.VMEM((K,), jnp.int32),
        pltpu.VMEM((K,), jnp.int32),
        pltpu.VMEM((K, D), jnp.float32),
        pltpu.SemaphoreType.DMA,
    ]
    if with_deg:
        out_type.append(jax.ShapeDtypeStruct((NC, NP, D), jnp.float32))
        scratch.append(pltpu.VMEM((K, D), jnp.float32))

    def prologue(z_hbm, agg_sh):
        c = lax.axis_index("c")
        s = lax.axis_index("s")
        wid = c * NS + s
        row0 = s * RPT
        pltpu.sync_copy(z_hbm, agg_sh.at[pl.ds(row0, RPT)])
        return c, s, wid, row0

    def chunk_loop(h_hbm, src_hbm, dst_hbm, wid,
                   src_v, dst_v, rows_v, sem, per_chunk):
        @pl.loop(0, CH)
        def _(j):
            base = pl.multiple_of(wid * EPW + j * K, 8)
            pltpu.sync_copy(src_hbm.at[pl.ds(base, K)], src_v)
            pltpu.sync_copy(dst_hbm.at[pl.ds(base, K)], dst_v)
            pltpu.async_copy(h_hbm.at[src_v], rows_v, sem).wait()
            per_chunk(rows_v, dst_v)

    def writeback(agg_sh, agg_hbm, c, row0):
        plsc.subcore_barrier()
        pltpu.sync_copy(agg_sh.at[pl.ds(row0, RPT)],
                        agg_hbm.at[c, pl.ds(row0, RPT)])

    if with_deg:
        od = jnp.ones((K, D), jnp.float32)

        @functools.partial(pl.kernel, mesh=_sc_mesh(), out_type=out_type,
                           scratch_types=scratch,
                           compiler_params=_sc_params())
        def body(h_hbm, src_hbm, dst_hbm, z_hbm, od_hbm,
                 agg_hbm, deg_hbm,
                 agg_sh, src_v, dst_v, rows_v, sem, ones_v):
            c, s, wid, row0 = prologue(z_hbm, agg_sh)
            pltpu.sync_copy(od_hbm, ones_v)
            plsc.subcore_barrier()

            @pl.loop(0, CH)
            def _(j):
                base = pl.multiple_of(wid * EPW + j * K, 8)
                pltpu.sync_copy(dst_hbm.at[pl.ds(base, K)], dst_v)
                pltpu.sync_copy(ones_v, agg_sh.at[dst_v], add=True)

            writeback(agg_sh, deg_hbm, c, row0)
            pltpu.sync_copy(z_hbm, agg_sh.at[pl.ds(row0, RPT)])
            plsc.subcore_barrier()

            def per_chunk(rows, didx):
                pltpu.sync_copy(rows, agg_sh.at[didx], add=True)

            chunk_loop(h_hbm, src_hbm, dst_hbm, wid,
                       src_v, dst_v, rows_v, sem, per_chunk)
            writeback(agg_sh, agg_hbm, c, row0)

        return body(h, src, dst3, z, od)

    @functools.partial(pl.kernel, mesh=_sc_mesh(), out_type=out_type,
                       scratch_types=scratch,
                       compiler_params=_sc_params())
    def body(h_hbm, src_hbm, dst_hbm, z_hbm,
             agg_hbm,
             agg_sh, src_v, dst_v, rows_v, sem):
        c, s, wid, row0 = prologue(z_hbm, agg_sh)
        plsc.subcore_barrier()

        def per_chunk(rows, didx):
            pltpu.sync_copy(rows, agg_sh.at[didx], add=True)

        chunk_loop(h_hbm, src_hbm, dst_hbm, wid,
                   src_v, dst_v, rows_v, sem, per_chunk)
        writeback(agg_sh, agg_hbm, c, row0)

    return (body(h, src, dst3, z)[0], None)


def kernel(content, edge_index, emb, W_proj, b_proj, W0, b0, W1, b1):
    src = edge_index[0]
    dst3 = edge_index[1]
    pad = ((0, NP - N), (0, 0))
    content_p = jnp.pad(content, pad)
    emb1_p = jnp.pad(lax.slice_in_dim(emb, 1, N + 1, axis=0), pad)
    h0 = _embed(content_p, emb1_p, W_proj, b_proj.reshape(1, D))
    agg0, deg = _sc_scatter(h0, src, dst3, with_deg=True)
    h1 = _layer(h0, agg0, deg, W0, b0.reshape(1, D), relu=True)
    agg1, _ = _sc_scatter(h1, src, dst3, with_deg=False)
    out = _layer(h1, agg1, deg, W1, b1.reshape(1, D), relu=False)
    return out[:N]


# speedup vs baseline: 6.7672x; 1.4431x over previous
import dataclasses
import functools

import jax
import jax.numpy as jnp
from jax import lax
from jax.experimental import pallas as pl
from jax.experimental.pallas import tpu as pltpu
from jax.experimental.pallas import tpu_sc as plsc

N = 10000
E = 320000
D = 128
NC = 2
NS = 16
NW = NC * NS
EPW = E // NW
K = 80
CH = EPW // K
SEG = 5
EPS = EPW // SEG
CHS = CH // SEG
NP = 10240
RPT = NP // NS
RB = 1024
GRID = NP // RB
DW = 16


def _leaky(x):
    return jnp.where(x >= 0, x, 0.01 * x)


def _embed_body(c_ref, e_ref, wp_ref, bp_ref, o_ref):
    y = lax.dot_general(c_ref[...], wp_ref[...], (((1,), (1,)), ((), ())),
                        preferred_element_type=jnp.float32) + bp_ref[...]
    o_ref[...] = e_ref[...] + _leaky(y)


def _embed(content_p, emb1_p, wp, bp):
    return pl.pallas_call(
        _embed_body,
        grid=(GRID,),
        in_specs=[
            pl.BlockSpec((RB, D), lambda i: (i, 0)),
            pl.BlockSpec((RB, D), lambda i: (i, 0)),
            pl.BlockSpec((D, D), lambda i: (0, 0)),
            pl.BlockSpec((1, D), lambda i: (0, 0)),
        ],
        out_specs=pl.BlockSpec((RB, D), lambda i: (i, 0)),
        out_shape=jax.ShapeDtypeStruct((NP, D), jnp.float32),
    )(content_p, emb1_p, wp, bp)


def _layer_body(relu, h_ref, agg_ref, deg_ref, w_ref, b_ref, o_ref):
    s = agg_ref[0] + agg_ref[1]
    dcol = (deg_ref[0] + deg_ref[1])[:, 0:1]
    inv = 1.0 / jnp.maximum(dcol, 1.0)
    h = h_ref[...]
    h_agg = s * inv
    w = w_ref[...]
    y = lax.dot_general(h, w[:, :D], (((1,), (1,)), ((), ())),
                        preferred_element_type=jnp.float32)
    y = y + lax.dot_general(h_agg, w[:, D:], (((1,), (1,)), ((), ())),
                            preferred_element_type=jnp.float32)
    y = y + b_ref[...]
    if relu:
        y = _leaky(y)
    nrm = jnp.maximum(jnp.sqrt(jnp.sum(y * y, axis=1, keepdims=True)), 1e-6)
    o_ref[...] = y / nrm


def _layer(h, agg, deg, w, b, relu):
    return pl.pallas_call(
        functools.partial(_layer_body, relu),
        grid=(GRID,),
        in_specs=[
            pl.BlockSpec((RB, D), lambda i: (i, 0)),
            pl.BlockSpec((NC, RB, D), lambda i: (0, i, 0)),
            pl.BlockSpec((NC, RB, D), lambda i: (0, i, 0)),
            pl.BlockSpec((D, 2 * D), lambda i: (0, 0)),
            pl.BlockSpec((1, D), lambda i: (0, 0)),
        ],
        out_specs=pl.BlockSpec((RB, D), lambda i: (i, 0)),
        out_shape=jax.ShapeDtypeStruct((NP, D), jnp.float32),
    )(h, agg, deg, w, b)


def _sc_mesh():
    return plsc.VectorSubcoreMesh(core_axis_name="c", subcore_axis_name="s")


def _sc_params():
    return pltpu.CompilerParams()


def _sc_scatter(h, src, dst3, with_deg):
    z = jnp.zeros((RPT, D), jnp.float32)

    out_type = [jax.ShapeDtypeStruct((NC, NP, D), jnp.float32)]
    scratch = [
        pltpu.VMEM_SHARED((NP, D), jnp.float32),
        pltpu.VMEM((K,), jnp.int32),
        pltpu.VMEM((K,), jnp.int32),
        pltpu.VMEM((K, D), jnp.float32),
        pltpu.SemaphoreType.DMA,
        pltpu.VMEM((K,), jnp.int32),
        pltpu.VMEM((K,), jnp.int32),
        pltpu.VMEM((K, D), jnp.float32),
        pltpu.SemaphoreType.DMA,
    ]
    if with_deg:
        out_type.append(jax.ShapeDtypeStruct((NC, NP, D), jnp.float32))

    def prologue(z_hbm, agg_sh):
        c = lax.axis_index("c")
        s = lax.axis_index("s")
        wid = c * NS + s
        row0 = s * RPT
        pltpu.sync_copy(z_hbm, agg_sh.at[pl.ds(row0, RPT)])
        return c, s, wid, row0

    def chunk_loop(h_hbm, src_hbm, dst_hbm, wid,
                   sa, da, ra, ma, sb, db, rb, mb, per_chunk):
        def stage_idx(buf_s, buf_d, j):
            base = pl.multiple_of(wid * EPW + j * K, 8)
            pltpu.sync_copy(src_hbm.at[pl.ds(base, K)], buf_s)
            pltpu.sync_copy(dst_hbm.at[pl.ds(base, K)], buf_d)

        stage_idx(sa, da, 0)
        pltpu.async_copy(h_hbm.at[sa], ra, ma)
        stage_idx(sb, db, 1)
        pltpu.async_copy(h_hbm.at[sb], rb, mb)

        @pl.loop(0, CH - 1, step=2)
        def _(j):
            pltpu.make_async_copy(h_hbm.at[sa], ra, ma).wait()
            per_chunk(ra, da)
            stage_idx(sa, da, j + 2)
            pltpu.async_copy(h_hbm.at[sa], ra, ma)
            pltpu.make_async_copy(h_hbm.at[sb], rb, mb).wait()
            per_chunk(rb, db)

            @pl.when(j + 3 < CH)
            def _():
                stage_idx(sb, db, j + 3)
                pltpu.async_copy(h_hbm.at[sb], rb, mb)

        pltpu.make_async_copy(h_hbm.at[sa], ra, ma).wait()
        per_chunk(ra, da)

    def writeback(agg_sh, agg_hbm, c, row0):
        plsc.subcore_barrier()
        pltpu.sync_copy(agg_sh.at[pl.ds(row0, RPT)],
                        agg_hbm.at[c, pl.ds(row0, RPT)])

    if with_deg:
        od = jnp.ones((K, D), jnp.float32)

        @functools.partial(pl.kernel, mesh=_sc_mesh(), out_type=out_type,
                           scratch_types=scratch,
                           compiler_params=_sc_params())
        def body(h_hbm, src_hbm, dst_hbm, z_hbm, od_hbm,
                 agg_hbm, deg_hbm,
                 agg_sh, sa, da, ra, ma, sb, db, rb, mb):
            c, s, wid, row0 = prologue(z_hbm, agg_sh)
            pltpu.sync_copy(od_hbm, ra)
            plsc.subcore_barrier()

            @pl.loop(0, CH)
            def _(j):
                base = pl.multiple_of(wid * EPW + j * K, 8)
                pltpu.sync_copy(dst_hbm.at[pl.ds(base, K)], da)
                pltpu.sync_copy(ra, agg_sh.at[da], add=True)

            writeback(agg_sh, deg_hbm, c, row0)
            pltpu.sync_copy(z_hbm, agg_sh.at[pl.ds(row0, RPT)])
            plsc.subcore_barrier()

            def per_chunk(rows, didx):
                pltpu.sync_copy(rows, agg_sh.at[didx], add=True)

            chunk_loop(h_hbm, src_hbm, dst_hbm, wid,
                       sa, da, ra, ma, sb, db, rb, mb, per_chunk)
            writeback(agg_sh, agg_hbm, c, row0)

        return body(h, src, dst3, z, od)

    @functools.partial(pl.kernel, mesh=_sc_mesh(), out_type=out_type,
                       scratch_types=scratch,
                       compiler_params=_sc_params())
    def body(h_hbm, src_hbm, dst_hbm, z_hbm,
             agg_hbm,
             agg_sh, sa, da, ra, ma, sb, db, rb, mb):
        c, s, wid, row0 = prologue(z_hbm, agg_sh)
        plsc.subcore_barrier()

        def per_chunk(rows, didx):
            pltpu.sync_copy(rows, agg_sh.at[didx], add=True)

        chunk_loop(h_hbm, src_hbm, dst_hbm, wid,
                   sa, da, ra, ma, sb, db, rb, mb, per_chunk)
        writeback(agg_sh, agg_hbm, c, row0)

    return (body(h, src, dst3, z)[0], None)


def kernel(content, edge_index, emb, W_proj, b_proj, W0, b0, W1, b1):
    src = edge_index[0]
    dst3 = edge_index[1]
    pad = ((0, NP - N), (0, 0))
    content_p = jnp.pad(content, pad)
    emb1_p = jnp.pad(lax.slice_in_dim(emb, 1, N + 1, axis=0), pad)
    h0 = _embed(content_p, emb1_p, W_proj, b_proj.reshape(1, D))
    agg0, deg = _sc_scatter(h0, src, dst3, with_deg=True)
    h1 = _layer(h0, agg0, deg, W0, b0.reshape(1, D), relu=True)
    agg1, _ = _sc_scatter(h1, src, dst3, with_deg=False)
    out = _layer(h1, agg1, deg, W1, b1.reshape(1, D), relu=False)
    return out[:N]


# speedup vs baseline: 7.4299x; 1.0979x over previous
import dataclasses
import functools

import jax
import jax.numpy as jnp
from jax import lax
from jax.experimental import pallas as pl
from jax.experimental.pallas import tpu as pltpu
from jax.experimental.pallas import tpu_sc as plsc

N = 10000
E = 320000
D = 128
NC = 2
NS = 16
NW = NC * NS
EPW = E // NW
K = 80
CH = EPW // K
SEG = 5
EPS = EPW // SEG
CHS = CH // SEG
NP = 10240
RPT = NP // NS
RB = 1024
GRID = NP // RB
DW = 16


def _leaky(x):
    return jnp.where(x >= 0, x, 0.01 * x)


def _embed_body(c_ref, e_ref, wp_ref, bp_ref, o_ref):
    y = lax.dot_general(c_ref[...], wp_ref[...], (((1,), (1,)), ((), ())),
                        preferred_element_type=jnp.float32) + bp_ref[...]
    o_ref[...] = e_ref[...] + _leaky(y)


def _embed(content_p, emb1_p, wp, bp):
    return pl.pallas_call(
        _embed_body,
        grid=(GRID,),
        in_specs=[
            pl.BlockSpec((RB, D), lambda i: (i, 0)),
            pl.BlockSpec((RB, D), lambda i: (i, 0)),
            pl.BlockSpec((D, D), lambda i: (0, 0)),
            pl.BlockSpec((1, D), lambda i: (0, 0)),
        ],
        out_specs=pl.BlockSpec((RB, D), lambda i: (i, 0)),
        out_shape=jax.ShapeDtypeStruct((NP, D), jnp.float32),
    )(content_p, emb1_p, wp, bp)


def _layer_body(relu, h_ref, agg_ref, deg_ref, w_ref, b_ref, o_ref):
    s = agg_ref[0] + agg_ref[1]
    dcol = (deg_ref[0] + deg_ref[1])[:, 0:1]
    inv = 1.0 / jnp.maximum(dcol, 1.0)
    h = h_ref[...]
    h_agg = s * inv
    w = w_ref[...]
    y = lax.dot_general(h, w[:, :D], (((1,), (1,)), ((), ())),
                        preferred_element_type=jnp.float32)
    y = y + lax.dot_general(h_agg, w[:, D:], (((1,), (1,)), ((), ())),
                            preferred_element_type=jnp.float32)
    y = y + b_ref[...]
    if relu:
        y = _leaky(y)
    nrm = jnp.maximum(jnp.sqrt(jnp.sum(y * y, axis=1, keepdims=True)), 1e-6)
    o_ref[...] = y / nrm


def _layer(h, agg, deg, w, b, relu):
    return pl.pallas_call(
        functools.partial(_layer_body, relu),
        grid=(GRID,),
        in_specs=[
            pl.BlockSpec((RB, D), lambda i: (i, 0)),
            pl.BlockSpec((NC, RB, D), lambda i: (0, i, 0)),
            pl.BlockSpec((NC, RB, D), lambda i: (0, i, 0)),
            pl.BlockSpec((D, 2 * D), lambda i: (0, 0)),
            pl.BlockSpec((1, D), lambda i: (0, 0)),
        ],
        out_specs=pl.BlockSpec((RB, D), lambda i: (i, 0)),
        out_shape=jax.ShapeDtypeStruct((NP, D), jnp.float32),
    )(h, agg, deg, w, b)


def _sc_mesh():
    return plsc.VectorSubcoreMesh(core_axis_name="c", subcore_axis_name="s")


def _sc_params():
    return pltpu.CompilerParams()


def _sc_scatter(h, src, dst3, with_deg):
    z = jnp.zeros((RPT, D), jnp.float32)

    out_type = [jax.ShapeDtypeStruct((NC, NP, D), jnp.float32)]
    scratch = [
        pltpu.VMEM_SHARED((NP, D), jnp.float32),
        pltpu.VMEM((K,), jnp.int32),
        pltpu.VMEM((K,), jnp.int32),
        pltpu.VMEM((K, D), jnp.float32),
        pltpu.SemaphoreType.DMA,
        pltpu.VMEM((K,), jnp.int32),
        pltpu.VMEM((K,), jnp.int32),
        pltpu.VMEM((K, D), jnp.float32),
        pltpu.SemaphoreType.DMA,
    ]
    if with_deg:
        out_type.append(jax.ShapeDtypeStruct((NC, NP, D), jnp.float32))

    def prologue(z_hbm, agg_sh):
        c = lax.axis_index("c")
        s = lax.axis_index("s")
        wid = c * NS + s
        row0 = s * RPT
        pltpu.sync_copy(z_hbm, agg_sh.at[pl.ds(row0, RPT)])
        return c, s, wid, row0

    def chunk_loop(h_hbm, src_hbm, dst_hbm, wid,
                   sa, da, ra, ma, sb, db, rb, mb, per_chunk):
        def stage_idx(buf_s, buf_d, j):
            base = pl.multiple_of(wid * EPW + j * K, 8)
            pltpu.sync_copy(src_hbm.at[pl.ds(base, K)], buf_s)
            pltpu.sync_copy(dst_hbm.at[pl.ds(base, K)], buf_d)

        stage_idx(sa, da, 0)
        pltpu.async_copy(h_hbm.at[sa], ra, ma)
        stage_idx(sb, db, 1)
        pltpu.async_copy(h_hbm.at[sb], rb, mb)

        @pl.loop(0, CH - 1, step=2)
        def _(j):
            pltpu.make_async_copy(h_hbm.at[sa], ra, ma).wait()
            per_chunk(ra, da)
            stage_idx(sa, da, j + 2)
            pltpu.async_copy(h_hbm.at[sa], ra, ma)
            pltpu.make_async_copy(h_hbm.at[sb], rb, mb).wait()
            per_chunk(rb, db)

            @pl.when(j + 3 < CH)
            def _():
                stage_idx(sb, db, j + 3)
                pltpu.async_copy(h_hbm.at[sb], rb, mb)

        pltpu.make_async_copy(h_hbm.at[sa], ra, ma).wait()
        per_chunk(ra, da)

    def writeback(agg_sh, agg_hbm, c, row0):
        plsc.subcore_barrier()
        pltpu.sync_copy(agg_sh.at[pl.ds(row0, RPT)],
                        agg_hbm.at[c, pl.ds(row0, RPT)])

    if with_deg:
        od = jnp.ones((K, D), jnp.float32)

        @functools.partial(pl.kernel, mesh=_sc_mesh(), out_type=out_type,
                           scratch_types=scratch,
                           compiler_params=_sc_params())
        def body(h_hbm, src_hbm, dst_hbm, z_hbm, od_hbm,
                 agg_hbm, deg_hbm,
                 agg_sh, sa, da, ra, ma, sb, db, rb, mb):
            c, s, wid, row0 = prologue(z_hbm, agg_sh)
            pltpu.sync_copy(od_hbm, ra)
            plsc.subcore_barrier()

            def dchunk(buf, m):
                base = pl.multiple_of(wid * EPW + m * K, 8)
                return dst_hbm.at[pl.ds(base, K)], buf

            pltpu.async_copy(*dchunk(da, 0), ma)

            @pl.loop(0, CH - 1, step=2)
            def _(j):
                pltpu.make_async_copy(*dchunk(da, 0), ma).wait()
                pltpu.async_copy(*dchunk(db, j + 1), mb)
                pltpu.sync_copy(ra, agg_sh.at[da], add=True)
                pltpu.make_async_copy(*dchunk(db, 0), mb).wait()
                pltpu.async_copy(*dchunk(da, j + 2), ma)
                pltpu.sync_copy(ra, agg_sh.at[db], add=True)

            pltpu.make_async_copy(*dchunk(da, 0), ma).wait()
            pltpu.sync_copy(ra, agg_sh.at[da], add=True)

            writeback(agg_sh, deg_hbm, c, row0)
            pltpu.sync_copy(z_hbm, agg_sh.at[pl.ds(row0, RPT)])
            plsc.subcore_barrier()

            def per_chunk(rows, didx):
                pltpu.sync_copy(rows, agg_sh.at[didx], add=True)

            chunk_loop(h_hbm, src_hbm, dst_hbm, wid,
                       sa, da, ra, ma, sb, db, rb, mb, per_chunk)
            writeback(agg_sh, agg_hbm, c, row0)

        return body(h, src, dst3, z, od)

    @functools.partial(pl.kernel, mesh=_sc_mesh(), out_type=out_type,
                       scratch_types=scratch,
                       compiler_params=_sc_params())
    def body(h_hbm, src_hbm, dst_hbm, z_hbm,
             agg_hbm,
             agg_sh, sa, da, ra, ma, sb, db, rb, mb):
        c, s, wid, row0 = prologue(z_hbm, agg_sh)
        plsc.subcore_barrier()

        def per_chunk(rows, didx):
            pltpu.sync_copy(rows, agg_sh.at[didx], add=True)

        chunk_loop(h_hbm, src_hbm, dst_hbm, wid,
                   sa, da, ra, ma, sb, db, rb, mb, per_chunk)
        writeback(agg_sh, agg_hbm, c, row0)

    return (body(h, src, dst3, z)[0], None)


def kernel(content, edge_index, emb, W_proj, b_proj, W0, b0, W1, b1):
    src = edge_index[0]
    dst3 = edge_index[1]
    pad = ((0, NP - N), (0, 0))
    content_p = jnp.pad(content, pad)
    emb1_p = jnp.pad(lax.slice_in_dim(emb, 1, N + 1, axis=0), pad)
    h0 = _embed(content_p, emb1_p, W_proj, b_proj.reshape(1, D))
    agg0, deg = _sc_scatter(h0, src, dst3, with_deg=True)
    h1 = _layer(h0, agg0, deg, W0, b0.reshape(1, D), relu=True)
    agg1, _ = _sc_scatter(h1, src, dst3, with_deg=False)
    out = _layer(h1, agg1, deg, W1, b1.reshape(1, D), relu=False)
    return out[:N]
